# baseline (speedup 1.0000x reference)
import jax
import jax.numpy as jnp
from jax.experimental import pallas as pl

_D = 2048
_N_IN = 8
_N_PROC = 64
_N_OUT = 8
_K = 8
_TILE = 2048


def _softmax0(s):
    m = jnp.max(s, axis=0, keepdims=True)
    e = jnp.exp(s - m)
    return e / jnp.sum(e, axis=0, keepdims=True)


def _router_body(x_ref, w_ref, idx_ref, pw_ref, iw_ref, ow_ref):
    s = jax.lax.dot_general(
        w_ref[...], x_ref[...],
        (((1,), (1,)), ((), ())),
        preferred_element_type=jnp.float32,
    )

    iw_ref[...] = _softmax0(s[_N_PROC:_N_PROC + _N_IN, :]).T
    ow_ref[...] = _softmax0(s[_N_PROC + _N_IN:_N_PROC + _N_IN + _N_OUT, :]).T

    sp = s[:_N_PROC, :]
    iota = jax.lax.broadcasted_iota(jnp.int32, sp.shape, 0)
    work = sp
    vals = []
    idxs = []
    for _ in range(_K):
        m = jnp.max(work, axis=0, keepdims=True)
        am = jnp.min(jnp.where(work == m, iota, _N_PROC), axis=0, keepdims=True)
        vals.append(m)
        idxs.append(am)
        work = jnp.where(iota == am, -jnp.inf, work)
    topv = jnp.concatenate(vals, axis=0)
    idx_ref[...] = jnp.concatenate(idxs, axis=0).T
    e = jnp.exp(topv - vals[0])
    pw_ref[...] = (e / jnp.sum(e, axis=0, keepdims=True)).T


@jax.jit
def kernel(x, W_in, W_proc, W_out):
    B, S, D = x.shape
    T = B * S
    xf = x.reshape(T, D)
    w_cat = jnp.concatenate([W_proc, W_in, W_out], axis=0)
    w_pad = jnp.pad(w_cat, ((0, 128 - w_cat.shape[0]), (0, 0)))

    grid = (T // _TILE,)
    idx, pw, iw, ow = pl.pallas_call(
        _router_body,
        grid=grid,
        in_specs=[
            pl.BlockSpec((_TILE, D), lambda i: (i, 0)),
            pl.BlockSpec((128, D), lambda i: (0, 0)),
        ],
        out_specs=[
            pl.BlockSpec((_TILE, _K), lambda i: (i, 0)),
            pl.BlockSpec((_TILE, _K), lambda i: (i, 0)),
            pl.BlockSpec((_TILE, _N_IN), lambda i: (i, 0)),
            pl.BlockSpec((_TILE, _N_OUT), lambda i: (i, 0)),
        ],
        out_shape=[
            jax.ShapeDtypeStruct((T, _K), jnp.int32),
            jax.ShapeDtypeStruct((T, _K), jnp.float32),
            jax.ShapeDtypeStruct((T, _N_IN), jnp.float32),
            jax.ShapeDtypeStruct((T, _N_OUT), jnp.float32),
        ],
    )(xf, w_pad)

    return (
        idx.reshape(B, S, _K),
        pw.reshape(B, S, _K),
        iw.reshape(B, S, _N_IN),
        ow.reshape(B, S, _N_OUT),
    )


# speedup vs baseline: 1.6211x; 1.6211x over previous
import jax
import jax.numpy as jnp
from jax.experimental import pallas as pl

_D = 2048
_N_IN = 8
_N_PROC = 64
_N_OUT = 8
_K = 8
_TILE = 2048


def _softmax0(s):
    m = jnp.max(s, axis=0, keepdims=True)
    e = jnp.exp(s - m)
    return e / jnp.sum(e, axis=0, keepdims=True)


def _router_body(x_ref, w_ref, idx_ref, pw_ref, iw_ref, ow_ref):
    s = jax.lax.dot_general(
        w_ref[...], x_ref[...],
        (((1,), (1,)), ((), ())),
        preferred_element_type=jnp.float32,
    )

    iw_ref[...] = _softmax0(s[_N_PROC:_N_PROC + _N_IN, :])
    ow_ref[...] = _softmax0(s[_N_PROC + _N_IN:_N_PROC + _N_IN + _N_OUT, :])

    sp = s[:_N_PROC, :]
    iota = jax.lax.broadcasted_iota(jnp.int32, sp.shape, 0)
    work = sp
    vals = []
    idxs = []
    for _ in range(_K):
        m = jnp.max(work, axis=0, keepdims=True)
        am = jnp.min(jnp.where(work == m, iota, _N_PROC), axis=0, keepdims=True)
        vals.append(m)
        idxs.append(am)
        work = jnp.where(iota == am, -jnp.inf, work)
    topv = jnp.concatenate(vals, axis=0)
    idx_ref[...] = jnp.concatenate(idxs, axis=0)
    e = jnp.exp(topv - vals[0])
    pw_ref[...] = e / jnp.sum(e, axis=0, keepdims=True)


@jax.jit
def kernel(x, W_in, W_proc, W_out):
    B, S, D = x.shape
    T = B * S
    xf = x.reshape(T, D)
    w_cat = jnp.concatenate([W_proc, W_in, W_out], axis=0)
    w_pad = jnp.pad(w_cat, ((0, 128 - w_cat.shape[0]), (0, 0)))

    grid = (T // _TILE,)
    idx, pw, iw, ow = pl.pallas_call(
        _router_body,
        grid=grid,
        in_specs=[
            pl.BlockSpec((_TILE, D), lambda i: (i, 0)),
            pl.BlockSpec((128, D), lambda i: (0, 0)),
        ],
        out_specs=[
            pl.BlockSpec((_K, _TILE), lambda i: (0, i)),
            pl.BlockSpec((_K, _TILE), lambda i: (0, i)),
            pl.BlockSpec((_N_IN, _TILE), lambda i: (0, i)),
            pl.BlockSpec((_N_OUT, _TILE), lambda i: (0, i)),
        ],
        out_shape=[
            jax.ShapeDtypeStruct((_K, T), jnp.int32),
            jax.ShapeDtypeStruct((_K, T), jnp.float32),
            jax.ShapeDtypeStruct((_N_IN, T), jnp.float32),
            jax.ShapeDtypeStruct((_N_OUT, T), jnp.float32),
        ],
    )(xf, w_pad)

    return (
        idx.T.reshape(B, S, _K),
        pw.T.reshape(B, S, _K),
        iw.T.reshape(B, S, _N_IN),
        ow.T.reshape(B, S, _N_OUT),
    )


# speedup vs baseline: 1.6569x; 1.0221x over previous
import jax
import jax.numpy as jnp
from jax.experimental import pallas as pl

_D = 2048
_DH = 1024
_N_IN = 8
_N_PROC = 64
_N_OUT = 8
_K = 8
_TILE = 2048


def _softmax0(s):
    m = jnp.max(s, axis=0, keepdims=True)
    e = jnp.exp(s - m)
    return e / jnp.sum(e, axis=0, keepdims=True)


def _router_body(xa_ref, xb_ref, wa_ref, wb_ref, idx_ref, pw_ref, iw_ref, ow_ref):
    dn = (((1,), (1,)), ((), ()))
    s = jax.lax.dot_general(wa_ref[...], xa_ref[...], dn,
                            preferred_element_type=jnp.float32)
    s = s + jax.lax.dot_general(wb_ref[...], xb_ref[...], dn,
                                preferred_element_type=jnp.float32)

    iw_ref[...] = _softmax0(s[_N_PROC:_N_PROC + _N_IN, :])
    ow_ref[...] = _softmax0(s[_N_PROC + _N_IN:_N_PROC + _N_IN + _N_OUT, :])

    sp = s[:_N_PROC, :]
    iota = jax.lax.broadcasted_iota(jnp.int32, sp.shape, 0)
    work = sp
    vals = []
    idxs = []
    for _ in range(_K):
        m = jnp.max(work, axis=0, keepdims=True)
        am = jnp.min(jnp.where(work == m, iota, _N_PROC), axis=0, keepdims=True)
        vals.append(m)
        idxs.append(am)
        work = jnp.where(iota == am, -jnp.inf, work)
    topv = jnp.concatenate(vals, axis=0)
    idx_ref[...] = jnp.concatenate(idxs, axis=0)
    e = jnp.exp(topv - vals[0])
    pw_ref[...] = e / jnp.sum(e, axis=0, keepdims=True)


@jax.jit
def kernel(x, W_in, W_proc, W_out):
    B, S, D = x.shape
    T = B * S
    xf = x.reshape(T, D)
    w_cat = jnp.concatenate([W_proc, W_in, W_out], axis=0)
    w_pad = jnp.pad(w_cat, ((0, 128 - w_cat.shape[0]), (0, 0)))

    grid = (T // _TILE,)
    idx, pw, iw, ow = pl.pallas_call(
        _router_body,
        grid=grid,
        in_specs=[
            pl.BlockSpec((_TILE, _DH), lambda i: (i, 0)),
            pl.BlockSpec((_TILE, _DH), lambda i: (i, 1)),
            pl.BlockSpec((128, _DH), lambda i: (0, 0)),
            pl.BlockSpec((128, _DH), lambda i: (0, 1)),
        ],
        out_specs=[
            pl.BlockSpec((_K, _TILE), lambda i: (0, i)),
            pl.BlockSpec((_K, _TILE), lambda i: (0, i)),
            pl.BlockSpec((_N_IN, _TILE), lambda i: (0, i)),
            pl.BlockSpec((_N_OUT, _TILE), lambda i: (0, i)),
        ],
        out_shape=[
            jax.ShapeDtypeStruct((_K, T), jnp.int32),
            jax.ShapeDtypeStruct((_K, T), jnp.float32),
            jax.ShapeDtypeStruct((_N_IN, T), jnp.float32),
            jax.ShapeDtypeStruct((_N_OUT, T), jnp.float32),
        ],
    )(xf, xf, w_pad, w_pad)

    return (
        idx.T.reshape(B, S, _K),
        pw.T.reshape(B, S, _K),
        iw.T.reshape(B, S, _N_IN),
        ow.T.reshape(B, S, _N_OUT),
    )
